# baseline (speedup 1.0000x reference)
import functools
import jax
import jax.numpy as jnp
from jax import lax
from jax.experimental import pallas as pl
from jax.experimental.pallas import tpu as pltpu
from jax.experimental.pallas import tpu_sc as plsc

CUTOFF = 5.0
GAMMA = 10.0


def _table_body(n0_ref, n1_ref, w0_ref, b0_ref, w1_ref, t_ref):
    x0 = jnp.dot(n0_ref[...], w0_ref[...], preferred_element_type=jnp.float32)
    parts = [x0 + b0_ref[...]]
    for a in range(3):
        parts.append(jnp.dot(n1_ref[a], w1_ref[...],
                             preferred_element_type=jnp.float32))
    h = jnp.concatenate(parts, axis=1)
    half = h.shape[1] // 2
    hu = lax.bitcast_convert_type(h[:, :half], jnp.uint32)
    lu = lax.bitcast_convert_type(h[:, half:], jnp.uint32)
    packed = (((hu + 0x8000) & jnp.uint32(0xFFFF0000))
              | ((lu + 0x8000) >> 16))
    t_ref[...] = lax.bitcast_convert_type(packed, jnp.int32)


def _build_table(n0, n1p, w0, b0, w1, bn):
    n = n0.shape[0]
    c_out = w0.shape[1]
    width = 2 * c_out
    grid = n // bn
    return pl.pallas_call(
        _table_body,
        grid=(grid,),
        in_specs=[
            pl.BlockSpec((bn, n0.shape[1]), lambda i: (i, 0)),
            pl.BlockSpec((3, bn, n0.shape[1]), lambda i: (0, i, 0)),
            pl.BlockSpec(w0.shape, lambda i: (0, 0)),
            pl.BlockSpec((1, c_out), lambda i: (0, 0)),
            pl.BlockSpec(w1.shape, lambda i: (0, 0)),
        ],
        out_specs=pl.BlockSpec((bn, width), lambda i: (i, 0)),
        out_shape=jax.ShapeDtypeStruct((n, width), jnp.int32),
    )(n0, n1p, w0, b0, w1)


CHUNK = 128


def _make_edge_gather(num_edges, width):
    info = plsc.get_sparse_core_info()
    nc, ns = info.num_cores, info.num_subcores
    nw = nc * ns
    n_tiles = num_edges // CHUNK
    base_steps = n_tiles // nw
    extra = n_tiles - base_steps * nw
    mesh = plsc.VectorSubcoreMesh(core_axis_name="c", subcore_axis_name="s")

    @functools.partial(
        pl.kernel,
        out_type=(
            jax.ShapeDtypeStruct((num_edges, width), jnp.int32),
            jax.ShapeDtypeStruct((8, num_edges), jnp.float32),
        ),
        mesh=mesh,
        scratch_types=[
            [pltpu.VMEM((CHUNK,), jnp.int32) for _ in range(3)],
            [pltpu.VMEM((CHUNK,), jnp.int32) for _ in range(3)],
            [pltpu.VMEM((CHUNK, width), jnp.int32) for _ in range(3)],
            [[pltpu.VMEM((CHUNK,), jnp.float32) for _ in range(6)]
             for _ in range(3)],
            [pltpu.VMEM((8, CHUNK), jnp.float32) for _ in range(3)],
            [pltpu.SemaphoreType.DMA for _ in range(3)],
            [pltpu.SemaphoreType.DMA for _ in range(3)],
        ],
        compiler_params=pltpu.CompilerParams(needs_layout_passes=False),
    )
    def edge_gather(table_hbm, cx_hbm, cy_hbm, cz_hbm, idxi_hbm, idxj_hbm,
                    gj_hbm, rijt_hbm,
                    idxi_v, idxj_v, rows_v, cvs, tb_v, sems, wsems):
        wid = lax.axis_index("s") * nc + lax.axis_index("c")
        zero16 = jnp.zeros((16,), jnp.float32)
        for b in range(3):
            for r in range(3, 8):
                for g in range(CHUNK // 16):
                    tb_v[b][r, pl.ds(g * 16, 16)] = zero16

        def copies(b):
            return [
                pltpu.make_async_copy(table_hbm.at[idxj_v[b]], rows_v[b],
                                      sems[b]),
                pltpu.make_async_copy(cx_hbm.at[idxi_v[b]], cvs[b][0],
                                      sems[b]),
                pltpu.make_async_copy(cy_hbm.at[idxi_v[b]], cvs[b][1],
                                      sems[b]),
                pltpu.make_async_copy(cz_hbm.at[idxi_v[b]], cvs[b][2],
                                      sems[b]),
                pltpu.make_async_copy(cx_hbm.at[idxj_v[b]], cvs[b][3],
                                      sems[b]),
                pltpu.make_async_copy(cy_hbm.at[idxj_v[b]], cvs[b][4],
                                      sems[b]),
                pltpu.make_async_copy(cz_hbm.at[idxj_v[b]], cvs[b][5],
                                      sems[b]),
            ]

        def start(t, b):
            base = t * CHUNK
            pltpu.sync_copy(idxj_hbm.at[pl.ds(base, CHUNK)], idxj_v[b])
            pltpu.sync_copy(idxi_hbm.at[pl.ds(base, CHUNK)], idxi_v[b])
            for cp in copies(b):
                cp.start()

        def wcopies(t, b):
            base = t * CHUNK
            return [
                pltpu.make_async_copy(rows_v[b], gj_hbm.at[pl.ds(base, CHUNK)],
                                      wsems[b]),
                pltpu.make_async_copy(tb_v[b],
                                      rijt_hbm.at[:, pl.ds(base, CHUNK)],
                                      wsems[b]),
            ]

        def finish(t, b):
            for cp in copies(b):
                cp.wait()
            for g in range(CHUNK // 16):
                s = pl.ds(g * 16, 16)
                tb_v[b][0, s] = cvs[b][3][s] - cvs[b][0][s]
                tb_v[b][1, s] = cvs[b][4][s] - cvs[b][1][s]
                tb_v[b][2, s] = cvs[b][5][s] - cvs[b][2][s]
            for cp in wcopies(t, b):
                cp.start()

        def wait_writes(b):
            for cp in wcopies(0, b):
                cp.wait()

        tile = lambda i: wid + i * nw
        assert base_steps % 3 == 0 and base_steps >= 6
        triples = base_steps // 3
        start(tile(0), 0)
        start(tile(1), 1)
        finish(tile(0), 0)
        start(tile(2), 2)
        finish(tile(1), 1)
        wait_writes(0)
        start(tile(3), 0)
        finish(tile(2), 2)
        wait_writes(1)
        start(tile(4), 1)

        def triple_body(tt, carry):
            i0 = 3 * tt

            def step(j, b, bn):
                finish(tile(i0 + j), b)

                @pl.when(i0 + j + 2 < base_steps)
                def _s():
                    wait_writes(bn)
                    start(tile(i0 + j + 2), bn)

            step(0, 0, 2)
            step(1, 1, 0)
            step(2, 2, 1)
            return carry

        lax.fori_loop(1, triples, triple_body, 0)
        wait_writes(0)
        wait_writes(1)
        wait_writes(2)

        @pl.when(wid < extra)
        def _tail():
            t = wid + base_steps * nw
            start(t, 0)
            finish(t, 0)
            wait_writes(0)

    return edge_gather


def _dot00(a, b):
    return lax.dot_general(a, b, (((0,), (0,)), ((), ())),
                           preferred_element_type=jnp.float32)


def _edge_body(gj_ref, rijt_ref, wr0_ref, wr1_ref, br0_ref, br1_ref,
               out0_ref, out1_ref):
    c_out = wr0_ref.shape[1]
    gt = lax.bitcast_convert_type(jnp.transpose(gj_ref[...]),
                                  jnp.uint32)
    hi = lax.bitcast_convert_type(gt & jnp.uint32(0xFFFF0000), jnp.float32)
    lo = lax.bitcast_convert_type(gt << 16, jnp.float32)
    g0t = hi[0:c_out, :]
    g1 = (hi[c_out:2 * c_out, :],
          lo[0:c_out, :],
          lo[c_out:2 * c_out, :])
    rijt = rijt_ref[...]
    be = rijt.shape[1]
    d2 = jnp.sum(rijt * rijt, axis=0, keepdims=True) + 1e-6
    d = jnp.sqrt(d2)
    rinv = 1.0 / d
    centers = lax.broadcasted_iota(jnp.int32, (16, be), 0).astype(
        jnp.float32) * (CUTOFF / 15.0)
    delta = d - centers
    rbf = jnp.exp(-GAMMA * delta * delta)
    fn0 = _dot00(wr0_ref[...], rbf) + br0_ref[...]
    fn1 = _dot00(wr1_ref[...], rbf) + br1_ref[...]
    acc = None
    for a in range(3):
        u_a = rijt[a:a + 1, :] * rinv
        out1_ref[pl.ds(a * c_out, c_out), :] = (g0t * u_a * fn1
                                                + g1[a] * fn0)
        ga_ua = g1[a] * u_a
        acc = ga_ua if acc is None else acc + ga_ua
    out0_ref[...] = g0t * fn0 + acc * fn1


def _edge_stage(gj, rijt, wr0, wr1, br0c, br1c, be):
    e = gj.shape[0]
    width = gj.shape[1]
    c_out = wr0.shape[1]
    grid = e // be
    full = lambda a: pl.BlockSpec(a.shape, lambda i: (0, 0))
    return pl.pallas_call(
        _edge_body,
        grid=(grid,),
        in_specs=[
            pl.BlockSpec((be, width), lambda i: (i, 0)),
            pl.BlockSpec((8, be), lambda i: (0, i)),
            full(wr0), full(wr1), full(br0c), full(br1c),
        ],
        out_specs=[
            pl.BlockSpec((c_out, be), lambda i: (0, i)),
            pl.BlockSpec((3 * c_out, be), lambda i: (0, i)),
        ],
        out_shape=[
            jax.ShapeDtypeStruct((c_out, e), jnp.float32),
            jax.ShapeDtypeStruct((3 * c_out, e), jnp.float32),
        ],
    )(gj, rijt, wr0, wr1, br0c, br1c)


def kernel(node_0, node_1, coord, idx_i, idx_j, W0, b0, W1, Wr0, br0, Wr1,
           br1):
    n, c_in = node_0.shape
    c_out = W0.shape[1]
    e = idx_i.shape[0]

    n1p = jnp.transpose(node_1, (2, 0, 1))
    coordt = jnp.transpose(coord)
    br0c = br0[:, None]
    br1c = br1[:, None]
    idxj32 = idx_j.astype(jnp.int32)
    idxi32 = idx_i.astype(jnp.int32)

    table = _build_table(node_0, n1p, W0, b0[None, :], W1, bn=1000)

    gj, rijt = _make_edge_gather(e, 2 * c_out)(
        table, coordt[0], coordt[1], coordt[2], idxi32, idxj32)

    out0t, out1t = _edge_stage(gj, rijt, Wr0, Wr1, br0c, br1c, be=6400)
    out0 = out0t.T
    out1 = out1t.reshape(3, c_out, e).transpose(2, 1, 0)
    return out0, out1


# speedup vs baseline: 1.0086x; 1.0086x over previous
import functools
import jax
import jax.numpy as jnp
from jax import lax
from jax.experimental import pallas as pl
from jax.experimental.pallas import tpu as pltpu
from jax.experimental.pallas import tpu_sc as plsc

CUTOFF = 5.0
GAMMA = 10.0


def _table_body(n0_ref, n1_ref, w0_ref, b0_ref, w1_ref, t_ref):
    x0 = jnp.dot(n0_ref[...], w0_ref[...], preferred_element_type=jnp.float32)
    parts = [x0 + b0_ref[...]]
    for a in range(3):
        parts.append(jnp.dot(n1_ref[a], w1_ref[...],
                             preferred_element_type=jnp.float32))
    h = jnp.concatenate(parts, axis=1)
    half = h.shape[1] // 2
    hu = lax.bitcast_convert_type(h[:, :half], jnp.uint32)
    lu = lax.bitcast_convert_type(h[:, half:], jnp.uint32)
    packed = (((hu + 0x8000) & jnp.uint32(0xFFFF0000))
              | ((lu + 0x8000) >> 16))
    t_ref[...] = lax.bitcast_convert_type(packed, jnp.int32)


def _build_table(n0, n1p, w0, b0, w1, bn):
    n = n0.shape[0]
    c_out = w0.shape[1]
    width = 2 * c_out
    grid = n // bn
    return pl.pallas_call(
        _table_body,
        grid=(grid,),
        in_specs=[
            pl.BlockSpec((bn, n0.shape[1]), lambda i: (i, 0)),
            pl.BlockSpec((3, bn, n0.shape[1]), lambda i: (0, i, 0)),
            pl.BlockSpec(w0.shape, lambda i: (0, 0)),
            pl.BlockSpec((1, c_out), lambda i: (0, 0)),
            pl.BlockSpec(w1.shape, lambda i: (0, 0)),
        ],
        out_specs=pl.BlockSpec((bn, width), lambda i: (i, 0)),
        out_shape=jax.ShapeDtypeStruct((n, width), jnp.int32),
    )(n0, n1p, w0, b0, w1)


CHUNK = 128


def _make_edge_gather(num_edges, width):
    info = plsc.get_sparse_core_info()
    nc, ns = info.num_cores, info.num_subcores
    nw = nc * ns
    n_tiles = num_edges // CHUNK
    base_steps = n_tiles // nw
    extra = n_tiles - base_steps * nw
    mesh = plsc.VectorSubcoreMesh(core_axis_name="c", subcore_axis_name="s")

    @functools.partial(
        pl.kernel,
        out_type=(
            jax.ShapeDtypeStruct((num_edges, width), jnp.int32),
            jax.ShapeDtypeStruct((8, num_edges), jnp.float32),
        ),
        mesh=mesh,
        scratch_types=[
            [pltpu.VMEM((CHUNK,), jnp.int32) for _ in range(2)],
            [pltpu.VMEM((CHUNK,), jnp.int32) for _ in range(2)],
            [pltpu.VMEM((CHUNK, width), jnp.int32) for _ in range(2)],
            [[pltpu.VMEM((CHUNK,), jnp.float32) for _ in range(6)]
             for _ in range(2)],
            [pltpu.VMEM((8, CHUNK), jnp.float32) for _ in range(2)],
            [pltpu.SemaphoreType.DMA for _ in range(2)],
        ],
        compiler_params=pltpu.CompilerParams(needs_layout_passes=False),
    )
    def edge_gather(table_hbm, cx_hbm, cy_hbm, cz_hbm, idxi_hbm, idxj_hbm,
                    gj_hbm, rijt_hbm,
                    idxi_v, idxj_v, rows_v, cvs, tb_v, sems):
        wid = lax.axis_index("s") * nc + lax.axis_index("c")
        zero16 = jnp.zeros((16,), jnp.float32)
        for b in range(2):
            for r in range(3, 8):
                for g in range(CHUNK // 16):
                    tb_v[b][r, pl.ds(g * 16, 16)] = zero16

        def copies(b):
            return [
                pltpu.make_async_copy(table_hbm.at[idxj_v[b]], rows_v[b],
                                      sems[b]),
                pltpu.make_async_copy(cx_hbm.at[idxi_v[b]], cvs[b][0],
                                      sems[b]),
                pltpu.make_async_copy(cy_hbm.at[idxi_v[b]], cvs[b][1],
                                      sems[b]),
                pltpu.make_async_copy(cz_hbm.at[idxi_v[b]], cvs[b][2],
                                      sems[b]),
                pltpu.make_async_copy(cx_hbm.at[idxj_v[b]], cvs[b][3],
                                      sems[b]),
                pltpu.make_async_copy(cy_hbm.at[idxj_v[b]], cvs[b][4],
                                      sems[b]),
                pltpu.make_async_copy(cz_hbm.at[idxj_v[b]], cvs[b][5],
                                      sems[b]),
            ]

        def start(t, b):
            base = t * CHUNK
            pltpu.sync_copy(idxj_hbm.at[pl.ds(base, CHUNK)], idxj_v[b])
            pltpu.sync_copy(idxi_hbm.at[pl.ds(base, CHUNK)], idxi_v[b])
            for cp in copies(b):
                cp.start()

        def finish(t, b):
            base = t * CHUNK
            for cp in copies(b):
                cp.wait()
            for g in range(CHUNK // 16):
                s = pl.ds(g * 16, 16)
                tb_v[b][0, s] = cvs[b][3][s] - cvs[b][0][s]
                tb_v[b][1, s] = cvs[b][4][s] - cvs[b][1][s]
                tb_v[b][2, s] = cvs[b][5][s] - cvs[b][2][s]
            pltpu.sync_copy(rows_v[b], gj_hbm.at[pl.ds(base, CHUNK)])
            pltpu.sync_copy(tb_v[b], rijt_hbm.at[:, pl.ds(base, CHUNK)])

        pairs = base_steps // 2
        rem = base_steps % 2
        tile = lambda i: wid + i * nw
        start(tile(0), 0)

        def pair_body(p, carry):
            i = 2 * p
            start(tile(i + 1), 1)
            finish(tile(i), 0)
            if rem == 1:
                start(tile(i + 2), 0)
            else:
                @pl.when(i + 2 < base_steps)
                def _s():
                    start(tile(i + 2), 0)
            finish(tile(i + 1), 1)
            return carry

        lax.fori_loop(0, pairs, pair_body, 0)
        if rem == 1:
            finish(tile(base_steps - 1), 0)

        @pl.when(wid < extra)
        def _tail():
            t = wid + base_steps * nw
            start(t, 0)
            finish(t, 0)

    return edge_gather


def _dot00(a, b):
    return lax.dot_general(a, b, (((0,), (0,)), ((), ())),
                           preferred_element_type=jnp.float32)


def _edge_body(gj_ref, rijt_ref, wr0_ref, wr1_ref, br0_ref, br1_ref,
               out0_ref, out1_ref):
    c_out = wr0_ref.shape[1]
    gt = lax.bitcast_convert_type(jnp.transpose(gj_ref[...]),
                                  jnp.uint32)
    hi = lax.bitcast_convert_type(gt & jnp.uint32(0xFFFF0000), jnp.float32)
    lo = lax.bitcast_convert_type(gt << 16, jnp.float32)
    g0t = hi[0:c_out, :]
    g1 = (hi[c_out:2 * c_out, :],
          lo[0:c_out, :],
          lo[c_out:2 * c_out, :])
    rijt = rijt_ref[...]
    be = rijt.shape[1]
    d2 = jnp.sum(rijt * rijt, axis=0, keepdims=True) + 1e-6
    d = jnp.sqrt(d2)
    rinv = 1.0 / d
    centers = lax.broadcasted_iota(jnp.int32, (16, be), 0).astype(
        jnp.float32) * (CUTOFF / 15.0)
    delta = d - centers
    rbf = jnp.exp(-GAMMA * delta * delta)
    fn0 = _dot00(wr0_ref[...], rbf) + br0_ref[...]
    fn1 = _dot00(wr1_ref[...], rbf) + br1_ref[...]
    acc = None
    for a in range(3):
        u_a = rijt[a:a + 1, :] * rinv
        out1_ref[pl.ds(a * c_out, c_out), :] = (g0t * u_a * fn1
                                                + g1[a] * fn0)
        ga_ua = g1[a] * u_a
        acc = ga_ua if acc is None else acc + ga_ua
    out0_ref[...] = g0t * fn0 + acc * fn1


def _edge_stage(gj, rijt, wr0, wr1, br0c, br1c, be):
    e = gj.shape[0]
    width = gj.shape[1]
    c_out = wr0.shape[1]
    grid = e // be
    full = lambda a: pl.BlockSpec(a.shape, lambda i: (0, 0))
    return pl.pallas_call(
        _edge_body,
        grid=(grid,),
        in_specs=[
            pl.BlockSpec((be, width), lambda i: (i, 0)),
            pl.BlockSpec((8, be), lambda i: (0, i)),
            full(wr0), full(wr1), full(br0c), full(br1c),
        ],
        out_specs=[
            pl.BlockSpec((c_out, be), lambda i: (0, i)),
            pl.BlockSpec((3 * c_out, be), lambda i: (0, i)),
        ],
        out_shape=[
            jax.ShapeDtypeStruct((c_out, e), jnp.float32),
            jax.ShapeDtypeStruct((3 * c_out, e), jnp.float32),
        ],
    )(gj, rijt, wr0, wr1, br0c, br1c)


def kernel(node_0, node_1, coord, idx_i, idx_j, W0, b0, W1, Wr0, br0, Wr1,
           br1):
    n, c_in = node_0.shape
    c_out = W0.shape[1]
    e = idx_i.shape[0]

    n1p = jnp.transpose(node_1, (2, 0, 1))
    coordt = jnp.transpose(coord)
    br0c = br0[:, None]
    br1c = br1[:, None]
    idxj32 = idx_j.astype(jnp.int32)
    idxi32 = idx_i.astype(jnp.int32)

    table = _build_table(node_0, n1p, W0, b0[None, :], W1, bn=1000)

    gj, rijt = _make_edge_gather(e, 2 * c_out)(
        table, coordt[0], coordt[1], coordt[2], idxi32, idxj32)

    out0t, out1t = _edge_stage(gj, rijt, Wr0, Wr1, br0c, br1c, be=6400)
    out0 = out0t.T
    out1 = out1t.reshape(3, c_out, e).transpose(2, 1, 0)
    return out0, out1
